# Initial kernel scaffold; baseline (speedup 1.0000x reference)
import jax
import jax.numpy as jnp
from jax.experimental import pallas as pl


def kernel(x, target):
    raise NotImplementedError("write your pallas kernel here")


# speedup vs baseline: 2.3945x; 2.3945x over previous
import functools
import math

import jax
import jax.numpy as jnp
from jax import lax
from jax.experimental import pallas as pl
from jax.experimental.pallas import tpu as pltpu
from jax.experimental.pallas import tpu_sc as plsc

SIZE = 32000
PADDING_IDX = 0
SMOOTHING = 0.1
CONFIDENCE = 1.0 - SMOOTHING
SMOOTH_VAL = SMOOTHING / (SIZE - 2)
K_ROW = CONFIDENCE * math.log(CONFIDENCE) + SMOOTH_VAL * (SIZE - 2) * math.log(SMOOTH_VAL)

N = 4096
ROW_BLOCK = 64
NUM_BLOCKS = N // ROW_BLOCK

SC_CORES = 2
SC_SUBCORES = 16
SC_WORKERS = SC_CORES * SC_SUBCORES
SC_CHUNK = N // SC_WORKERS
SC_LANES = 16


def _sc_gather_body(x_flat_hbm, tgt_hbm, out_hbm, tgt_v, flat_v, vals_v, acc_v, sem):
    wid = lax.axis_index("s") * SC_CORES + lax.axis_index("c")
    base = wid * SC_CHUNK
    pltpu.sync_copy(tgt_hbm.at[pl.ds(base, SC_CHUNK)], tgt_v)
    for k in range(SC_CHUNK // SC_LANES):
        t = tgt_v[pl.ds(k * SC_LANES, SC_LANES)]
        rows = (base + k * SC_LANES) + lax.broadcasted_iota(jnp.int32, (SC_LANES,), 0)
        flat_v[pl.ds(k * SC_LANES, SC_LANES)] = rows * SIZE + t
    pltpu.async_copy(x_flat_hbm.at[flat_v], vals_v, sem).wait()
    acc = jnp.zeros((SC_LANES,), jnp.float32)
    for k in range(SC_CHUNK // SC_LANES):
        t = tgt_v[pl.ds(k * SC_LANES, SC_LANES)]
        v = vals_v[pl.ds(k * SC_LANES, SC_LANES)]
        acc = acc + jnp.where(t != PADDING_IDX, v, 0.0)
    acc_v[...] = acc
    pltpu.sync_copy(acc_v, out_hbm.at[wid])


@functools.cache
def _sc_gather():
    return pl.kernel(
        _sc_gather_body,
        mesh=plsc.VectorSubcoreMesh(core_axis_name="c", subcore_axis_name="s"),
        out_type=jax.ShapeDtypeStruct((SC_WORKERS, SC_LANES), jnp.float32),
        scratch_types=[
            pltpu.VMEM((SC_CHUNK,), jnp.int32),
            pltpu.VMEM((SC_CHUNK,), jnp.int32),
            pltpu.VMEM((SC_CHUNK,), jnp.float32),
            pltpu.VMEM((SC_LANES,), jnp.float32),
            pltpu.SemaphoreType.DMA,
        ],
    )


def _tc_body(x_ref, t_ref, g_ref, out_ref):
    i = pl.program_id(0)
    m = (t_ref[...] != PADDING_IDX).astype(jnp.float32)
    xb = x_ref[...]
    masked = xb * m
    contrib = (
        jnp.float32(K_ROW) * jnp.sum(m)
        - jnp.float32(SMOOTH_VAL) * jnp.sum(masked)
        + jnp.float32(SMOOTH_VAL) * jnp.sum(masked[:, 0:1])
    )

    @pl.when(i == 0)
    def _():
        out_ref[0, 0] = -jnp.float32(CONFIDENCE - SMOOTH_VAL) * jnp.sum(g_ref[...])

    out_ref[0, 0] += contrib


def _tc_reduce(x, t2d, g_parts):
    return pl.pallas_call(
        _tc_body,
        grid=(NUM_BLOCKS,),
        in_specs=[
            pl.BlockSpec((ROW_BLOCK, SIZE), lambda i: (i, 0)),
            pl.BlockSpec((ROW_BLOCK, 1), lambda i: (i, 0)),
            pl.BlockSpec((SC_WORKERS, SC_LANES), lambda i: (0, 0)),
        ],
        out_specs=pl.BlockSpec((1, 1), lambda i: (0, 0), memory_space=pltpu.SMEM),
        out_shape=jax.ShapeDtypeStruct((1, 1), jnp.float32),
        compiler_params=pltpu.CompilerParams(
            dimension_semantics=("arbitrary",),
        ),
    )(x, t2d, g_parts)


def kernel(x, target):
    target = target.astype(jnp.int32)
    g_parts = _sc_gather()(x.reshape(-1), target)
    out = _tc_reduce(x, target.reshape(N, 1), g_parts)
    return out[0, 0]


# speedup vs baseline: 2.5499x; 1.0649x over previous
import functools
import math

import jax
import jax.numpy as jnp
from jax import lax
from jax.experimental import pallas as pl
from jax.experimental.pallas import tpu as pltpu
from jax.experimental.pallas import tpu_sc as plsc

SIZE = 32000
PADDING_IDX = 0
SMOOTHING = 0.1
CONFIDENCE = 1.0 - SMOOTHING
SMOOTH_VAL = SMOOTHING / (SIZE - 2)
K_ROW = CONFIDENCE * math.log(CONFIDENCE) + SMOOTH_VAL * (SIZE - 2) * math.log(SMOOTH_VAL)

N = 4096
ROW_BLOCK = 128
NUM_BLOCKS = N // ROW_BLOCK

SC_CORES = 2
SC_SUBCORES = 16
SC_WORKERS = SC_CORES * SC_SUBCORES
SC_CHUNK = N // SC_WORKERS
SC_LANES = 16


def _sc_gather_body(x_flat_hbm, tgt_hbm, out_hbm, tgt_v, flat_v, vals_v, acc_v, sem):
    wid = lax.axis_index("s") * SC_CORES + lax.axis_index("c")
    base = wid * SC_CHUNK
    pltpu.sync_copy(tgt_hbm.at[pl.ds(base, SC_CHUNK)], tgt_v)
    for k in range(SC_CHUNK // SC_LANES):
        t = tgt_v[pl.ds(k * SC_LANES, SC_LANES)]
        rows = (base + k * SC_LANES) + lax.broadcasted_iota(jnp.int32, (SC_LANES,), 0)
        flat_v[pl.ds(k * SC_LANES, SC_LANES)] = rows * SIZE + t
    pltpu.async_copy(x_flat_hbm.at[flat_v], vals_v, sem).wait()
    acc = jnp.zeros((SC_LANES,), jnp.float32)
    for k in range(SC_CHUNK // SC_LANES):
        t = tgt_v[pl.ds(k * SC_LANES, SC_LANES)]
        v = vals_v[pl.ds(k * SC_LANES, SC_LANES)]
        acc = acc + jnp.where(t != PADDING_IDX, v, 0.0)
    acc_v[...] = acc
    pltpu.sync_copy(acc_v, out_hbm.at[wid])


@functools.cache
def _sc_gather():
    return pl.kernel(
        _sc_gather_body,
        mesh=plsc.VectorSubcoreMesh(core_axis_name="c", subcore_axis_name="s"),
        out_type=jax.ShapeDtypeStruct((SC_WORKERS, SC_LANES), jnp.float32),
        scratch_types=[
            pltpu.VMEM((SC_CHUNK,), jnp.int32),
            pltpu.VMEM((SC_CHUNK,), jnp.int32),
            pltpu.VMEM((SC_CHUNK,), jnp.float32),
            pltpu.VMEM((SC_LANES,), jnp.float32),
            pltpu.SemaphoreType.DMA,
        ],
    )


def _tc_body(x_ref, t_ref, g_ref, out_ref):
    i = pl.program_id(0)
    m = (t_ref[...] != PADDING_IDX).astype(jnp.float32)
    xb = x_ref[...]
    rowsums = jnp.sum(xb, axis=1, keepdims=True)
    contrib = (
        jnp.float32(K_ROW) * jnp.sum(m)
        - jnp.float32(SMOOTH_VAL) * jnp.sum(m * rowsums)
        + jnp.float32(SMOOTH_VAL) * jnp.sum(m * xb[:, 0:1])
    )

    @pl.when(i == 0)
    def _():
        out_ref[0, 0] = -jnp.float32(CONFIDENCE - SMOOTH_VAL) * jnp.sum(g_ref[...])

    out_ref[0, 0] += contrib


def _tc_reduce(x, t2d, g_parts):
    return pl.pallas_call(
        _tc_body,
        grid=(NUM_BLOCKS,),
        in_specs=[
            pl.BlockSpec((ROW_BLOCK, SIZE), lambda i: (i, 0)),
            pl.BlockSpec((ROW_BLOCK, 1), lambda i: (i, 0)),
            pl.BlockSpec((SC_WORKERS, SC_LANES), lambda i: (0, 0)),
        ],
        out_specs=pl.BlockSpec((1, 1), lambda i: (0, 0), memory_space=pltpu.SMEM),
        out_shape=jax.ShapeDtypeStruct((1, 1), jnp.float32),
        compiler_params=pltpu.CompilerParams(
            dimension_semantics=("arbitrary",),
        ),
    )(x, t2d, g_parts)


def kernel(x, target):
    target = target.astype(jnp.int32)
    g_parts = _sc_gather()(x.reshape(-1), target)
    out = _tc_reduce(x, target.reshape(N, 1), g_parts)
    return out[0, 0]
